# baseline (speedup 1.0000x reference)
import functools

import jax
import jax.numpy as jnp
from jax import lax
from jax.experimental import pallas as pl
from jax.experimental.pallas import tpu as pltpu
from jax.experimental.pallas import tpu_sc as plsc

VOCAB = 1_000_000
HID = 64
BATCH = 16384
LANES = 16

NUM_CORES = 2
NUM_SUBCORES = 16
NW = NUM_CORES * NUM_SUBCORES
BPW = BATCH // NW
CHUNK = 128
NCHUNK = BPW // CHUNK

_MESH = plsc.VectorSubcoreMesh(core_axis_name="c", subcore_axis_name="s")


@functools.partial(
    pl.kernel,
    out_type=jax.ShapeDtypeStruct((BATCH,), jnp.float32),
    mesh=_MESH,
    scratch_types=[
        pltpu.VMEM((BPW,), jnp.int32),
        pltpu.VMEM((BPW,), jnp.int32),
        pltpu.VMEM((BPW,), jnp.float32),
        pltpu.VMEM((BPW, HID), jnp.float32),
        pltpu.VMEM((BPW, HID), jnp.float32),
        pltpu.VMEM((BPW,), jnp.float32),
        pltpu.SemaphoreType.DMA,
    ],
    compiler_params=pltpu.CompilerParams(
        needs_layout_passes=False, use_tc_tiling_on_sc=False),
)
def _skipgram(center_hbm, target_hbm, label_hbm, in_hbm, outt_hbm, y_hbm,
              cidx, tidx, lab, crows, trows, yv, sem):
    wid = lax.axis_index("s") * NUM_CORES + lax.axis_index("c")
    base = wid * BPW

    pltpu.sync_copy(center_hbm.at[pl.ds(base, BPW)], cidx)
    pltpu.sync_copy(target_hbm.at[pl.ds(base, BPW)], tidx)
    pltpu.sync_copy(label_hbm.at[pl.ds(base, BPW)], lab)

    handles = []
    for j in range(NCHUNK):
        sl = pl.ds(j * CHUNK, CHUNK)
        handles.append(pltpu.async_copy(in_hbm.at[cidx.at[sl]], crows.at[sl], sem))
        handles.append(pltpu.async_copy(outt_hbm.at[tidx.at[sl]], trows.at[sl], sem))
    for h in handles:
        h.wait()

    lane = lax.iota(jnp.int32, LANES)
    onehot = [jnp.where(lane == r, 1.0, 0.0).astype(jnp.float32)
              for r in range(LANES)]

    def group_body(g, _):
        gbase = g * LANES
        labv = lab[pl.ds(gbase, LANES)]
        res = jnp.zeros((LANES,), jnp.float32)
        for r in range(LANES):
            row = gbase + r
            acc = crows[row, pl.ds(0, LANES)] * trows[row, pl.ds(0, LANES)]
            for k in range(1, HID // LANES):
                acc = acc + (crows[row, pl.ds(k * LANES, LANES)]
                             * trows[row, pl.ds(k * LANES, LANES)])
            res = res + jnp.sum(acc) * onehot[r]
        yv[pl.ds(gbase, LANES)] = res * labv
        return 0

    lax.fori_loop(0, BPW // LANES, group_body, 0)

    pltpu.sync_copy(yv, y_hbm.at[pl.ds(base, BPW)])


def kernel(center, target, label, in_emb, out_emb):
    center = center.astype(jnp.int32)
    target = target.astype(jnp.int32)
    return _skipgram(center, target, label, in_emb, out_emb)
